# baseline (speedup 1.0000x reference)
import functools
import jax
import jax.numpy as jnp
import numpy as np
from jax import lax
from jax.experimental import pallas as pl
from jax.experimental.pallas import tpu as pltpu
from jax.experimental.pallas import tpu_sc as plsc

AVG = float(np.log(33.0))
CSTD = float(np.sqrt(1e-5))
NGROUP = 64
NC, NS = 2, 16
NW = NC * NS
BWIN = 2560
UNR = 4


@functools.lru_cache(maxsize=None)
def _make_seg4(n, e, nf, cpw, gather):
    assert nf == NW * cpw and e % BWIN == 0 and n % 16 == 0
    assert gather or cpw == 1
    nwin = e // BWIN
    nvr = BWIN // 16
    mesh = plsc.VectorSubcoreMesh(core_axis_name="c", subcore_axis_name="s")
    out_sds = jax.ShapeDtypeStruct((nf * n,), jnp.float32)

    @functools.partial(
        pl.kernel,
        out_type=(out_sds,) * 4,
        mesh=mesh,
        compiler_params=pltpu.CompilerParams(needs_layout_passes=False),
        scratch_types=[
            pltpu.VMEM((cpw * n,) if gather else (BWIN,), jnp.float32),
            pltpu.VMEM((BWIN,), jnp.int32),
            pltpu.VMEM((BWIN,), jnp.int32),
            pltpu.VMEM((cpw * n,), jnp.float32),
            pltpu.VMEM((cpw * n,), jnp.float32),
            pltpu.VMEM((cpw * n,), jnp.float32),
            pltpu.VMEM((cpw * n,), jnp.float32),
            pltpu.VMEM((n,), jnp.int32),
        ],
    )
    def seg4(vals_hbm, src_hbm, dst_hbm, s_out, q_out, mn_out, mx_out,
             vals_v, srcv, dstv, s_acc, q_acc, mn_acc, mx_acc, tmp):
        cid = lax.axis_index("c")
        sid = lax.axis_index("s")
        wid = sid * NC + cid
        lanes = lax.iota(jnp.int32, 16)
        pw2 = jnp.left_shift(jnp.ones((16,), jnp.int32), lanes)
        if gather:
            pltpu.sync_copy(vals_hbm.at[pl.ds(wid * cpw * n, cpw * n)], vals_v)

        zero = jnp.zeros((16,), jnp.float32)
        pinf = jnp.full((16,), jnp.inf, jnp.float32)
        ninf = jnp.full((16,), -jnp.inf, jnp.float32)

        def initb(i, _):
            sl = pl.ds(i * 16, 16)
            s_acc[sl] = zero
            q_acc[sl] = zero
            mn_acc[sl] = pinf
            mx_acc[sl] = ninf
            return 0

        lax.fori_loop(0, cpw * n // 16, initb, 0)

        def wloop(w, _):
            if gather:
                pltpu.sync_copy(src_hbm.at[pl.ds(w * BWIN, BWIN)], srcv)
            else:
                pltpu.sync_copy(
                    vals_hbm.at[pl.ds(wid * e + w * BWIN, BWIN)], vals_v)
            pltpu.sync_copy(dst_hbm.at[pl.ds(w * BWIN, BWIN)], dstv)

            ids = [lanes + u * 16 for u in range(UNR)]

            def vrb(jb, _):
                base = jb * UNR * 16
                d_idx = [dstv[pl.ds(base + u * 16, 16)] for u in range(UNR)]
                if gather:
                    s_idx = [srcv[pl.ds(base + u * 16, 16)] for u in range(UNR)]
                else:
                    v0 = [vals_v[pl.ds(base + u * 16, 16)] for u in range(UNR)]

                def body(bits):
                    rems = [(jnp.full((16,), bits[u], jnp.int32) & pw2) != 0
                            for u in range(UNR)]
                    for u in range(UNR):
                        plsc.store_scatter(tmp, [d_idx[u]], ids[u], mask=rems[u])
                    wins = []
                    for u in range(UNR):
                        got = plsc.load_gather(tmp, [d_idx[u]])
                        wins.append(rems[u] & (got == ids[u]))
                    for u in range(UNR):
                        for r in range(cpw):
                            if gather:
                                v = plsc.load_gather(vals_v, [s_idx[u] + (r * n)])
                            else:
                                v = v0[u]
                            a = d_idx[u] + (r * n)
                            win = wins[u]
                            cs = plsc.load_gather(s_acc, [a])
                            plsc.store_scatter(s_acc, [a], cs + v, mask=win)
                            cq = plsc.load_gather(q_acc, [a])
                            plsc.store_scatter(q_acc, [a], cq + v * v, mask=win)
                            cm = plsc.load_gather(mn_acc, [a])
                            plsc.store_scatter(mn_acc, [a], jnp.minimum(cm, v), mask=win)
                            cx = plsc.load_gather(mx_acc, [a])
                            plsc.store_scatter(mx_acc, [a], jnp.maximum(cx, v), mask=win)
                    return tuple(
                        bits[u] & jnp.bitwise_not(jnp.sum(jnp.where(wins[u], pw2, 0)))
                        for u in range(UNR))

                bits1 = body((jnp.int32(0xFFFF),) * UNR)
                anyb = bits1[0]
                for u in range(1, UNR):
                    anyb = anyb | bits1[u]

                @pl.when(anyb != 0)
                def _():
                    lax.fori_loop(0, 15, lambda i, b: body(b), bits1)

                return 0

            lax.fori_loop(0, nvr // UNR, vrb, 0)
            return 0

        lax.fori_loop(0, nwin, wloop, 0)

        sl = pl.ds(wid * cpw * n, cpw * n)
        pltpu.sync_copy(s_acc, s_out.at[sl])
        pltpu.sync_copy(q_acc, q_out.at[sl])
        pltpu.sync_copy(mn_acc, mn_out.at[sl])
        pltpu.sync_copy(mx_acc, mx_out.at[sl])

    return seg4


def _x_stats(xT, src, dst, d):
    n = xT.shape[1]
    e = src.shape[0]
    if d < 128:
        xT = jnp.concatenate([xT, jnp.zeros((128 - d, n), jnp.float32)], axis=0)
    k = _make_seg4(n, e, 64, 2, True)
    outs = []
    for p in range(2):
        sl = xT[p * 64:(p + 1) * 64].reshape(-1)
        outs.append(k(sl, src, dst))
    res = []
    for i in range(4):
        full = jnp.concatenate(
            [outs[0][i].reshape(64, n), outs[1][i].reshape(64, n)], axis=0)
        res.append(full[:d].T)
    return res


def _head_body(x_ref, batch_ref, wl_ref, bl_ref, o_ref):
    x = x_ref[...]
    n = x.shape[0]
    b = batch_ref[...].reshape(n, 1)
    gids = jax.lax.broadcasted_iota(jnp.int32, (n, NGROUP), 1)
    onehot = (b == gids).astype(jnp.float32)
    cnt = jnp.sum(onehot, axis=0)
    pooled = jax.lax.dot_general(onehot, x, (((0,), (0,)), ((), ())))
    pooled = pooled / jnp.clip(cnt, 1.0, None)[:, None]
    out = pooled @ wl_ref[...] + bl_ref[...]
    out = out - jnp.max(out, axis=-1, keepdims=True)
    ex = jnp.exp(out)
    o_ref[...] = ex / jnp.sum(ex, axis=-1, keepdims=True)


def _head(x, batch, Wl, bl):
    return pl.pallas_call(
        _head_body,
        out_shape=jax.ShapeDtypeStruct((NGROUP, Wl.shape[1]), jnp.float32),
    )(x, batch, Wl, bl)


def kernel(x, edge_index, edge_attr, batch, W0, b0, W1, b1, g1, beta1, W2, b2, g2, beta2, W3, b3, g3, beta3, W4, b4, g4, beta4, Wl, bl):
    src, dst = edge_index[0], edge_index[1]
    n, f_in = x.shape
    e = src.shape[0]

    eaT = jnp.concatenate([
        edge_attr.T,
        jnp.ones((1, e), jnp.float32),
        jnp.zeros((NW - 17, e), jnp.float32),
    ], axis=0)
    ea_k = _make_seg4(n, e, NW, 1, False)
    ea_s, ea_q, ea_mn, ea_mx = (o.reshape(NW, n) for o in ea_k(eaT.reshape(-1), src, dst))
    count = ea_s[16]

    deg = jnp.clip(count, 1.0, None)
    inv = 1.0 / deg
    hasm = (count > 0)[:, None]
    dl = jnp.log(deg + 1.0)
    s1 = (dl / AVG)[:, None]
    s2 = (AVG / dl)[:, None]

    ea_mean = ea_s[:16].T * inv[:, None]
    ea_std = jnp.sqrt(jax.nn.relu(ea_q[:16].T * inv[:, None] - ea_mean**2) + 1e-5)
    ea_mn = jnp.where(hasm, ea_mn[:16].T, 0.0)
    ea_mx = jnp.where(hasm, ea_mx[:16].T, 0.0)
    ea_agg = jnp.concatenate([ea_mean, ea_mn, ea_mx, ea_std], axis=-1)

    def layer(xc, xcT, W, b):
        D = xc.shape[1]
        Dt = 2 * D + 16
        s, sq, mn, mx = _x_stats(xcT, src, dst, D)
        mean_src = s * inv[:, None]
        std_src = jnp.sqrt(jax.nn.relu(sq * inv[:, None] - mean_src**2) + 1e-5)
        mn = jnp.where(hasm, mn, 0.0)
        mx = jnp.where(hasm, mx, 0.0)
        xt = jnp.where(hasm, xc, 0.0)
        y = 0.0
        for k in range(3):
            Wk = W[k * 4 * Dt:(k + 1) * 4 * Dt]
            Wd = Wk[0:D] + Wk[Dt:Dt + D] + Wk[2 * Dt:2 * Dt + D]
            yk = xt @ Wd + CSTD * jnp.sum(Wk[3 * Dt:3 * Dt + D], axis=0)
            yk += mean_src @ Wk[D:2 * D] + mn @ Wk[Dt + D:Dt + 2 * D]
            yk += mx @ Wk[2 * Dt + D:2 * Dt + 2 * D] + std_src @ Wk[3 * Dt + D:3 * Dt + 2 * D]
            Wea = jnp.concatenate([Wk[2 * D:2 * D + 16], Wk[Dt + 2 * D:Dt + 2 * D + 16],
                                   Wk[2 * Dt + 2 * D:2 * Dt + 2 * D + 16], Wk[3 * Dt + 2 * D:3 * Dt + 2 * D + 16]], axis=0)
            yk += ea_agg @ Wea
            scale = 1.0 if k == 0 else (s1 if k == 1 else s2)
            y = y + yk * scale
        return y + b

    xc = layer(x, x.T, W0, b0)
    for W, bb, g, be in ((W1, b1, g1, beta1), (W2, b2, g2, beta2), (W3, b3, g3, beta3), (W4, b4, g4, beta4)):
        y = layer(xc, xc.T, W, bb)
        m = jnp.mean(y, axis=0)
        v = jnp.var(y, axis=0)
        h = jax.nn.relu((y - m) / jnp.sqrt(v + 1e-5) * g + be)
        xc = h + xc
    return _head(xc, batch, Wl, bl)


# speedup vs baseline: 2.4362x; 2.4362x over previous
import functools
import jax
import jax.numpy as jnp
import numpy as np
from jax import lax
from jax.experimental import pallas as pl
from jax.experimental.pallas import tpu as pltpu
from jax.experimental.pallas import tpu_sc as plsc

AVG = float(np.log(33.0))
CSTD = float(np.sqrt(1e-5))
NGROUP = 64
NC, NS = 2, 16
NW = NC * NS
BWIN = 2560
UNR = 4


@functools.lru_cache(maxsize=None)
def _make_seg4(n, e, nf, cpw, gather):
    assert nf == NW * cpw and e % BWIN == 0 and n % 16 == 0
    assert gather or cpw == 1
    nwin = e // BWIN
    nvr = BWIN // 16
    mesh = plsc.VectorSubcoreMesh(core_axis_name="c", subcore_axis_name="s")
    out_sds = jax.ShapeDtypeStruct((nf * n,), jnp.float32)

    @functools.partial(
        pl.kernel,
        out_type=(out_sds,) * 4,
        mesh=mesh,
        compiler_params=pltpu.CompilerParams(needs_layout_passes=False),
        scratch_types=[
            pltpu.VMEM((cpw * n,) if gather else (BWIN,), jnp.float32),
            pltpu.VMEM((BWIN,), jnp.int32),
            pltpu.VMEM((BWIN,), jnp.int32),
            pltpu.VMEM((cpw * n,), jnp.float32),
            pltpu.VMEM((cpw * n,), jnp.float32),
            pltpu.VMEM((cpw * n,), jnp.float32),
            pltpu.VMEM((cpw * n,), jnp.float32),
            pltpu.VMEM((n,), jnp.int32),
        ],
    )
    def seg4(vals_hbm, src_hbm, dst_hbm, s_out, q_out, mn_out, mx_out,
             vals_v, srcv, dstv, s_acc, q_acc, mn_acc, mx_acc, tmp):
        cid = lax.axis_index("c")
        sid = lax.axis_index("s")
        wid = sid * NC + cid
        lanes = lax.iota(jnp.int32, 16)
        pw2 = jnp.left_shift(jnp.ones((16,), jnp.int32), lanes)
        if gather:
            pltpu.sync_copy(vals_hbm.at[pl.ds(wid * cpw * n, cpw * n)], vals_v)

        zero = jnp.zeros((16,), jnp.float32)
        pinf = jnp.full((16,), jnp.inf, jnp.float32)
        ninf = jnp.full((16,), -jnp.inf, jnp.float32)

        def initb(i, _):
            sl = pl.ds(i * 16, 16)
            s_acc[sl] = zero
            q_acc[sl] = zero
            mn_acc[sl] = pinf
            mx_acc[sl] = ninf
            return 0

        lax.fori_loop(0, cpw * n // 16, initb, 0)

        def wloop(w, _):
            if gather:
                pltpu.sync_copy(src_hbm.at[pl.ds(w * BWIN, BWIN)], srcv)
            else:
                pltpu.sync_copy(
                    vals_hbm.at[pl.ds(wid * e + w * BWIN, BWIN)], vals_v)
            pltpu.sync_copy(dst_hbm.at[pl.ds(w * BWIN, BWIN)], dstv)

            ids = [lanes + u * 16 for u in range(UNR)]

            def vrb(jb, _):
                base = jb * UNR * 16
                d_idx = [dstv[pl.ds(base + u * 16, 16)] for u in range(UNR)]
                if gather:
                    s_idx = [srcv[pl.ds(base + u * 16, 16)] for u in range(UNR)]
                else:
                    v0 = [vals_v[pl.ds(base + u * 16, 16)] for u in range(UNR)]

                def body(bits):
                    rems = [(jnp.full((16,), bits[u], jnp.int32) & pw2) != 0
                            for u in range(UNR)]
                    for u in range(UNR):
                        plsc.store_scatter(tmp, [d_idx[u]], ids[u], mask=rems[u])
                    wins = []
                    for u in range(UNR):
                        got = plsc.load_gather(tmp, [d_idx[u]])
                        wins.append(rems[u] & (got == ids[u]))
                    for u in range(UNR):
                        for r in range(cpw):
                            if gather:
                                v = plsc.load_gather(vals_v, [s_idx[u] + (r * n)])
                            else:
                                v = v0[u]
                            a = d_idx[u] + (r * n)
                            win = wins[u]
                            cs = plsc.load_gather(s_acc, [a])
                            plsc.store_scatter(s_acc, [a], cs + v, mask=win)
                            cq = plsc.load_gather(q_acc, [a])
                            plsc.store_scatter(q_acc, [a], cq + v * v, mask=win)
                            cm = plsc.load_gather(mn_acc, [a])
                            plsc.store_scatter(mn_acc, [a], jnp.minimum(cm, v), mask=win)
                            cx = plsc.load_gather(mx_acc, [a])
                            plsc.store_scatter(mx_acc, [a], jnp.maximum(cx, v), mask=win)
                    return tuple(
                        bits[u] & jnp.bitwise_not(jnp.sum(jnp.where(wins[u], pw2, 0)))
                        for u in range(UNR))

                bits1 = body((jnp.int32(0xFFFF),) * UNR)
                anyb = bits1[0]
                for u in range(1, UNR):
                    anyb = anyb | bits1[u]

                @pl.when(anyb != 0)
                def _():
                    def retry(i, b):
                        run = b[0]
                        for u in range(1, UNR):
                            run = run | b[u]
                        return lax.cond(run != 0, body, lambda bb: bb, b)

                    lax.fori_loop(0, 15, retry, bits1)

                return 0

            lax.fori_loop(0, nvr // UNR, vrb, 0)
            return 0

        lax.fori_loop(0, nwin, wloop, 0)

        sl = pl.ds(wid * cpw * n, cpw * n)
        pltpu.sync_copy(s_acc, s_out.at[sl])
        pltpu.sync_copy(q_acc, q_out.at[sl])
        pltpu.sync_copy(mn_acc, mn_out.at[sl])
        pltpu.sync_copy(mx_acc, mx_out.at[sl])

    return seg4


def _x_stats(xT, src, dst, d):
    n = xT.shape[1]
    e = src.shape[0]
    if d < 128:
        xT = jnp.concatenate([xT, jnp.zeros((128 - d, n), jnp.float32)], axis=0)
    k = _make_seg4(n, e, 64, 2, True)
    outs = []
    for p in range(2):
        sl = xT[p * 64:(p + 1) * 64].reshape(-1)
        outs.append(k(sl, src, dst))
    res = []
    for i in range(4):
        full = jnp.concatenate(
            [outs[0][i].reshape(64, n), outs[1][i].reshape(64, n)], axis=0)
        res.append(full[:d].T)
    return res


def _head_body(x_ref, batch_ref, wl_ref, bl_ref, o_ref):
    x = x_ref[...]
    n = x.shape[0]
    b = batch_ref[...].reshape(n, 1)
    gids = jax.lax.broadcasted_iota(jnp.int32, (n, NGROUP), 1)
    onehot = (b == gids).astype(jnp.float32)
    cnt = jnp.sum(onehot, axis=0)
    pooled = jax.lax.dot_general(onehot, x, (((0,), (0,)), ((), ())))
    pooled = pooled / jnp.clip(cnt, 1.0, None)[:, None]
    out = pooled @ wl_ref[...] + bl_ref[...]
    out = out - jnp.max(out, axis=-1, keepdims=True)
    ex = jnp.exp(out)
    o_ref[...] = ex / jnp.sum(ex, axis=-1, keepdims=True)


def _head(x, batch, Wl, bl):
    return pl.pallas_call(
        _head_body,
        out_shape=jax.ShapeDtypeStruct((NGROUP, Wl.shape[1]), jnp.float32),
    )(x, batch, Wl, bl)


def kernel(x, edge_index, edge_attr, batch, W0, b0, W1, b1, g1, beta1, W2, b2, g2, beta2, W3, b3, g3, beta3, W4, b4, g4, beta4, Wl, bl):
    src, dst = edge_index[0], edge_index[1]
    n, f_in = x.shape
    e = src.shape[0]

    eaT = jnp.concatenate([
        edge_attr.T,
        jnp.ones((1, e), jnp.float32),
        jnp.zeros((NW - 17, e), jnp.float32),
    ], axis=0)
    ea_k = _make_seg4(n, e, NW, 1, False)
    ea_s, ea_q, ea_mn, ea_mx = (o.reshape(NW, n) for o in ea_k(eaT.reshape(-1), src, dst))
    count = ea_s[16]

    deg = jnp.clip(count, 1.0, None)
    inv = 1.0 / deg
    hasm = (count > 0)[:, None]
    dl = jnp.log(deg + 1.0)
    s1 = (dl / AVG)[:, None]
    s2 = (AVG / dl)[:, None]

    ea_mean = ea_s[:16].T * inv[:, None]
    ea_std = jnp.sqrt(jax.nn.relu(ea_q[:16].T * inv[:, None] - ea_mean**2) + 1e-5)
    ea_mn = jnp.where(hasm, ea_mn[:16].T, 0.0)
    ea_mx = jnp.where(hasm, ea_mx[:16].T, 0.0)
    ea_agg = jnp.concatenate([ea_mean, ea_mn, ea_mx, ea_std], axis=-1)

    def layer(xc, xcT, W, b):
        D = xc.shape[1]
        Dt = 2 * D + 16
        s, sq, mn, mx = _x_stats(xcT, src, dst, D)
        mean_src = s * inv[:, None]
        std_src = jnp.sqrt(jax.nn.relu(sq * inv[:, None] - mean_src**2) + 1e-5)
        mn = jnp.where(hasm, mn, 0.0)
        mx = jnp.where(hasm, mx, 0.0)
        xt = jnp.where(hasm, xc, 0.0)
        y = 0.0
        for k in range(3):
            Wk = W[k * 4 * Dt:(k + 1) * 4 * Dt]
            Wd = Wk[0:D] + Wk[Dt:Dt + D] + Wk[2 * Dt:2 * Dt + D]
            yk = xt @ Wd + CSTD * jnp.sum(Wk[3 * Dt:3 * Dt + D], axis=0)
            yk += mean_src @ Wk[D:2 * D] + mn @ Wk[Dt + D:Dt + 2 * D]
            yk += mx @ Wk[2 * Dt + D:2 * Dt + 2 * D] + std_src @ Wk[3 * Dt + D:3 * Dt + 2 * D]
            Wea = jnp.concatenate([Wk[2 * D:2 * D + 16], Wk[Dt + 2 * D:Dt + 2 * D + 16],
                                   Wk[2 * Dt + 2 * D:2 * Dt + 2 * D + 16], Wk[3 * Dt + 2 * D:3 * Dt + 2 * D + 16]], axis=0)
            yk += ea_agg @ Wea
            scale = 1.0 if k == 0 else (s1 if k == 1 else s2)
            y = y + yk * scale
        return y + b

    xc = layer(x, x.T, W0, b0)
    for W, bb, g, be in ((W1, b1, g1, beta1), (W2, b2, g2, beta2), (W3, b3, g3, beta3), (W4, b4, g4, beta4)):
        y = layer(xc, xc.T, W, bb)
        m = jnp.mean(y, axis=0)
        v = jnp.var(y, axis=0)
        h = jax.nn.relu((y - m) / jnp.sqrt(v + 1e-5) * g + be)
        xc = h + xc
    return _head(xc, batch, Wl, bl)


# speedup vs baseline: 2.4364x; 1.0001x over previous
import functools
import jax
import jax.numpy as jnp
import numpy as np
from jax import lax
from jax.experimental import pallas as pl
from jax.experimental.pallas import tpu as pltpu
from jax.experimental.pallas import tpu_sc as plsc

AVG = float(np.log(33.0))
CSTD = float(np.sqrt(1e-5))
NGROUP = 64
NC, NS = 2, 16
NW = NC * NS
BWIN = 2560
UNR = 4


@functools.lru_cache(maxsize=None)
def _make_seg4(n, e, nf, cpw, gather):
    assert nf == NW * cpw and e % BWIN == 0 and n % 16 == 0
    assert gather or cpw == 1
    nwin = e // BWIN
    nvr = BWIN // 16
    mesh = plsc.VectorSubcoreMesh(core_axis_name="c", subcore_axis_name="s")
    out_sds = jax.ShapeDtypeStruct((nf * n,), jnp.float32)

    @functools.partial(
        pl.kernel,
        out_type=(out_sds,) * 4,
        mesh=mesh,
        compiler_params=pltpu.CompilerParams(needs_layout_passes=False),
        scratch_types=(
            [pltpu.VMEM((n,) if gather else (BWIN,), jnp.float32)
             for _ in range(cpw if gather else 1)]
            + [pltpu.VMEM((BWIN,), jnp.int32)] * 2
            + [pltpu.VMEM((n,), jnp.float32) for _ in range(4 * cpw)]
            + [pltpu.VMEM((n,), jnp.int32)] * 2
        ),
    )
    def seg4(vals_hbm, src_hbm, dst_hbm, s_out, q_out, mn_out, mx_out, *refs):
        nv = cpw if gather else 1
        vals_vs = refs[:nv]
        srcv, dstv = refs[nv], refs[nv + 1]
        accs = refs[nv + 2:nv + 2 + 4 * cpw]
        tmps = refs[nv + 2 + 4 * cpw:]
        cid = lax.axis_index("c")
        sid = lax.axis_index("s")
        wid = sid * NC + cid
        lanes = lax.iota(jnp.int32, 16)
        pw2 = jnp.left_shift(jnp.ones((16,), jnp.int32), lanes)
        if gather:
            for r in range(cpw):
                pltpu.sync_copy(
                    vals_hbm.at[pl.ds((wid * cpw + r) * n, n)], vals_vs[r])

        zero = jnp.zeros((16,), jnp.float32)
        pinf = jnp.full((16,), jnp.inf, jnp.float32)
        ninf = jnp.full((16,), -jnp.inf, jnp.float32)
        inits = [zero, zero, pinf, ninf]

        def initb(i, _):
            sl = pl.ds(i * 16, 16)
            for st in range(4):
                for r in range(cpw):
                    accs[st * cpw + r][sl] = inits[st]
            return 0

        lax.fori_loop(0, n // 16, initb, 0)

        def wloop(w, _):
            if gather:
                pltpu.sync_copy(src_hbm.at[pl.ds(w * BWIN, BWIN)], srcv)
            else:
                pltpu.sync_copy(
                    vals_hbm.at[pl.ds(wid * e + w * BWIN, BWIN)], vals_vs[0])
            pltpu.sync_copy(dst_hbm.at[pl.ds(w * BWIN, BWIN)], dstv)

            ids = [lanes + u * 16 for u in range(UNR)]

            def block(jb, tmp):
                base = jb * UNR * 16
                d_idx = [dstv[pl.ds(base + u * 16, 16)] for u in range(UNR)]
                if gather:
                    s_idx = [srcv[pl.ds(base + u * 16, 16)] for u in range(UNR)]
                else:
                    v0 = [vals_vs[0][pl.ds(base + u * 16, 16)] for u in range(UNR)]

                def body(bits):
                    rems = [(jnp.full((16,), bits[u], jnp.int32) & pw2) != 0
                            for u in range(UNR)]
                    for u in range(UNR):
                        plsc.store_scatter(tmp, [d_idx[u]], ids[u], mask=rems[u])
                    wins = []
                    for u in range(UNR):
                        got = plsc.load_gather(tmp, [d_idx[u]])
                        wins.append(rems[u] & (got == ids[u]))
                    for u in range(UNR):
                        for r in range(cpw):
                            if gather:
                                v = plsc.load_gather(vals_vs[r], [s_idx[u]])
                            else:
                                v = v0[u]
                            a = d_idx[u]
                            win = wins[u]
                            ops = (lambda c: c + v, lambda c: c + v * v,
                                   lambda c: jnp.minimum(c, v),
                                   lambda c: jnp.maximum(c, v))
                            for st in range(4):
                                acc = accs[st * cpw + r]
                                c0 = plsc.load_gather(acc, [a])
                                plsc.store_scatter(acc, [a], ops[st](c0), mask=win)
                    return tuple(
                        bits[u] & jnp.bitwise_not(jnp.sum(jnp.where(wins[u], pw2, 0)))
                        for u in range(UNR))

                bits1 = body((jnp.int32(0xFFFF),) * UNR)
                anyb = bits1[0]
                for u in range(1, UNR):
                    anyb = anyb | bits1[u]

                @pl.when(anyb != 0)
                def _():
                    def retry(i, b):
                        run = b[0]
                        for u in range(1, UNR):
                            run = run | b[u]
                        return lax.cond(run != 0, body, lambda bb: bb, b)

                    lax.fori_loop(0, 15, retry, bits1)

            def vrb(jp, _):
                block(jp * 2, tmps[0])
                block(jp * 2 + 1, tmps[1])
                return 0

            lax.fori_loop(0, nvr // UNR // 2, vrb, 0)
            return 0

        lax.fori_loop(0, nwin, wloop, 0)

        for st, out in enumerate((s_out, q_out, mn_out, mx_out)):
            for r in range(cpw):
                pltpu.sync_copy(
                    accs[st * cpw + r],
                    out.at[pl.ds((wid * cpw + r) * n, n)])

    return seg4


def _x_stats(xT, src, dst, d):
    n = xT.shape[1]
    e = src.shape[0]
    if d < 128:
        xT = jnp.concatenate([xT, jnp.zeros((128 - d, n), jnp.float32)], axis=0)
    k = _make_seg4(n, e, 64, 2, True)
    outs = []
    for p in range(2):
        sl = xT[p * 64:(p + 1) * 64].reshape(-1)
        outs.append(k(sl, src, dst))
    res = []
    for i in range(4):
        full = jnp.concatenate(
            [outs[0][i].reshape(64, n), outs[1][i].reshape(64, n)], axis=0)
        res.append(full[:d].T)
    return res


def _head_body(x_ref, batch_ref, wl_ref, bl_ref, o_ref):
    x = x_ref[...]
    n = x.shape[0]
    b = batch_ref[...].reshape(n, 1)
    gids = jax.lax.broadcasted_iota(jnp.int32, (n, NGROUP), 1)
    onehot = (b == gids).astype(jnp.float32)
    cnt = jnp.sum(onehot, axis=0)
    pooled = jax.lax.dot_general(onehot, x, (((0,), (0,)), ((), ())))
    pooled = pooled / jnp.clip(cnt, 1.0, None)[:, None]
    out = pooled @ wl_ref[...] + bl_ref[...]
    out = out - jnp.max(out, axis=-1, keepdims=True)
    ex = jnp.exp(out)
    o_ref[...] = ex / jnp.sum(ex, axis=-1, keepdims=True)


def _head(x, batch, Wl, bl):
    return pl.pallas_call(
        _head_body,
        out_shape=jax.ShapeDtypeStruct((NGROUP, Wl.shape[1]), jnp.float32),
    )(x, batch, Wl, bl)


def kernel(x, edge_index, edge_attr, batch, W0, b0, W1, b1, g1, beta1, W2, b2, g2, beta2, W3, b3, g3, beta3, W4, b4, g4, beta4, Wl, bl):
    src, dst = edge_index[0], edge_index[1]
    n, f_in = x.shape
    e = src.shape[0]

    eaT = jnp.concatenate([
        edge_attr.T,
        jnp.ones((1, e), jnp.float32),
        jnp.zeros((NW - 17, e), jnp.float32),
    ], axis=0)
    ea_k = _make_seg4(n, e, NW, 1, False)
    ea_s, ea_q, ea_mn, ea_mx = (o.reshape(NW, n) for o in ea_k(eaT.reshape(-1), src, dst))
    count = ea_s[16]

    deg = jnp.clip(count, 1.0, None)
    inv = 1.0 / deg
    hasm = (count > 0)[:, None]
    dl = jnp.log(deg + 1.0)
    s1 = (dl / AVG)[:, None]
    s2 = (AVG / dl)[:, None]

    ea_mean = ea_s[:16].T * inv[:, None]
    ea_std = jnp.sqrt(jax.nn.relu(ea_q[:16].T * inv[:, None] - ea_mean**2) + 1e-5)
    ea_mn = jnp.where(hasm, ea_mn[:16].T, 0.0)
    ea_mx = jnp.where(hasm, ea_mx[:16].T, 0.0)
    ea_agg = jnp.concatenate([ea_mean, ea_mn, ea_mx, ea_std], axis=-1)

    def layer(xc, xcT, W, b):
        D = xc.shape[1]
        Dt = 2 * D + 16
        s, sq, mn, mx = _x_stats(xcT, src, dst, D)
        mean_src = s * inv[:, None]
        std_src = jnp.sqrt(jax.nn.relu(sq * inv[:, None] - mean_src**2) + 1e-5)
        mn = jnp.where(hasm, mn, 0.0)
        mx = jnp.where(hasm, mx, 0.0)
        xt = jnp.where(hasm, xc, 0.0)
        y = 0.0
        for k in range(3):
            Wk = W[k * 4 * Dt:(k + 1) * 4 * Dt]
            Wd = Wk[0:D] + Wk[Dt:Dt + D] + Wk[2 * Dt:2 * Dt + D]
            yk = xt @ Wd + CSTD * jnp.sum(Wk[3 * Dt:3 * Dt + D], axis=0)
            yk += mean_src @ Wk[D:2 * D] + mn @ Wk[Dt + D:Dt + 2 * D]
            yk += mx @ Wk[2 * Dt + D:2 * Dt + 2 * D] + std_src @ Wk[3 * Dt + D:3 * Dt + 2 * D]
            Wea = jnp.concatenate([Wk[2 * D:2 * D + 16], Wk[Dt + 2 * D:Dt + 2 * D + 16],
                                   Wk[2 * Dt + 2 * D:2 * Dt + 2 * D + 16], Wk[3 * Dt + 2 * D:3 * Dt + 2 * D + 16]], axis=0)
            yk += ea_agg @ Wea
            scale = 1.0 if k == 0 else (s1 if k == 1 else s2)
            y = y + yk * scale
        return y + b

    xc = layer(x, x.T, W0, b0)
    for W, bb, g, be in ((W1, b1, g1, beta1), (W2, b2, g2, beta2), (W3, b3, g3, beta3), (W4, b4, g4, beta4)):
        y = layer(xc, xc.T, W, bb)
        m = jnp.mean(y, axis=0)
        v = jnp.var(y, axis=0)
        h = jax.nn.relu((y - m) / jnp.sqrt(v + 1e-5) * g + be)
        xc = h + xc
    return _head(xc, batch, Wl, bl)


# speedup vs baseline: 2.8671x; 1.1768x over previous
import functools
import jax
import jax.numpy as jnp
import numpy as np
from jax import lax
from jax.experimental import pallas as pl
from jax.experimental.pallas import tpu as pltpu
from jax.experimental.pallas import tpu_sc as plsc

AVG = float(np.log(33.0))
CSTD = float(np.sqrt(1e-5))
NGROUP = 64
NC, NS = 2, 16
NW = NC * NS
BWIN = 2560
UNR = 4


@functools.lru_cache(maxsize=None)
def _make_seg4(n, e, nf, cpw, gather):
    assert nf == NW * cpw and e % BWIN == 0 and n % 16 == 0
    assert gather or cpw == 1
    nwin = e // BWIN
    nvr = BWIN // 16
    mesh = plsc.VectorSubcoreMesh(core_axis_name="c", subcore_axis_name="s")
    out_sds = jax.ShapeDtypeStruct((nf * n,), jnp.float32)

    @functools.partial(
        pl.kernel,
        out_type=(out_sds,) * 4,
        mesh=mesh,
        compiler_params=pltpu.CompilerParams(needs_layout_passes=False),
        scratch_types=(
            [pltpu.VMEM((n,) if gather else (BWIN,), jnp.float32)
             for _ in range(cpw if gather else 1)]
            + [pltpu.VMEM((BWIN,), jnp.int32)] * 2
            + [pltpu.VMEM((n,), jnp.float32) for _ in range(4 * cpw)]
            + [pltpu.VMEM((n,), jnp.int32)] * 2
        ),
    )
    def seg4(vals_hbm, src_hbm, dst_hbm, s_out, q_out, mn_out, mx_out, *refs):
        nv = cpw if gather else 1
        vals_vs = refs[:nv]
        srcv, dstv = refs[nv], refs[nv + 1]
        accs = refs[nv + 2:nv + 2 + 4 * cpw]
        tmps = refs[nv + 2 + 4 * cpw:]
        cid = lax.axis_index("c")
        sid = lax.axis_index("s")
        wid = sid * NC + cid
        lanes = lax.iota(jnp.int32, 16)
        pw2 = jnp.left_shift(jnp.ones((16,), jnp.int32), lanes)
        if gather:
            for r in range(cpw):
                pltpu.sync_copy(
                    vals_hbm.at[pl.ds((wid * cpw + r) * n, n)], vals_vs[r])

        zero = jnp.zeros((16,), jnp.float32)
        pinf = jnp.full((16,), jnp.inf, jnp.float32)
        ninf = jnp.full((16,), -jnp.inf, jnp.float32)
        inits = [zero, zero, pinf, ninf]

        def initb(i, _):
            sl = pl.ds(i * 16, 16)
            for st in range(4):
                for r in range(cpw):
                    accs[st * cpw + r][sl] = inits[st]
            return 0

        lax.fori_loop(0, n // 16, initb, 0)

        def wloop(w, _):
            if gather:
                pltpu.sync_copy(src_hbm.at[pl.ds(w * BWIN, BWIN)], srcv)
            else:
                pltpu.sync_copy(
                    vals_hbm.at[pl.ds(wid * e + w * BWIN, BWIN)], vals_vs[0])
            pltpu.sync_copy(dst_hbm.at[pl.ds(w * BWIN, BWIN)], dstv)

            ids = [lanes + u * 16 for u in range(UNR)]

            def block(jb, tmp):
                base = jb * UNR * 16
                d_idx = [dstv[pl.ds(base + u * 16, 16)] for u in range(UNR)]
                if gather:
                    s_idx = [srcv[pl.ds(base + u * 16, 16)] for u in range(UNR)]
                else:
                    v0 = [vals_vs[0][pl.ds(base + u * 16, 16)] for u in range(UNR)]

                def body(bits):
                    rems = [(jnp.full((16,), bits[u], jnp.int32) & pw2) != 0
                            for u in range(UNR)]
                    for u in range(UNR):
                        plsc.store_scatter(tmp, [d_idx[u]], ids[u], mask=rems[u])
                    wins = []
                    for u in range(UNR):
                        got = plsc.load_gather(tmp, [d_idx[u]])
                        wins.append(rems[u] & (got == ids[u]))
                    for u in range(UNR):
                        for r in range(cpw):
                            if gather:
                                v = plsc.load_gather(vals_vs[r], [s_idx[u]])
                            else:
                                v = v0[u]
                            a = d_idx[u]
                            win = wins[u]
                            plsc.addupdate_scatter(accs[r], [a], v, mask=win)
                            plsc.addupdate_scatter(accs[cpw + r], [a], v * v, mask=win)
                            for st in (2, 3):
                                acc = accs[st * cpw + r]
                                c0 = plsc.load_gather(acc, [a])
                                nv = jnp.minimum(c0, v) if st == 2 else jnp.maximum(c0, v)
                                plsc.store_scatter(acc, [a], nv, mask=win)
                    return tuple(
                        bits[u] & jnp.bitwise_not(jnp.sum(jnp.where(wins[u], pw2, 0)))
                        for u in range(UNR))

                bits1 = body((jnp.int32(0xFFFF),) * UNR)
                anyb = bits1[0]
                for u in range(1, UNR):
                    anyb = anyb | bits1[u]

                @pl.when(anyb != 0)
                def _():
                    def retry(i, b):
                        run = b[0]
                        for u in range(1, UNR):
                            run = run | b[u]
                        return lax.cond(run != 0, body, lambda bb: bb, b)

                    lax.fori_loop(0, 15, retry, bits1)

            def vrb(jp, _):
                block(jp * 2, tmps[0])
                block(jp * 2 + 1, tmps[1])
                return 0

            lax.fori_loop(0, nvr // UNR // 2, vrb, 0)
            return 0

        lax.fori_loop(0, nwin, wloop, 0)

        for st, out in enumerate((s_out, q_out, mn_out, mx_out)):
            for r in range(cpw):
                pltpu.sync_copy(
                    accs[st * cpw + r],
                    out.at[pl.ds((wid * cpw + r) * n, n)])

    return seg4


def _x_stats(xT, src, dst, d):
    n = xT.shape[1]
    e = src.shape[0]
    if d < 128:
        xT = jnp.concatenate([xT, jnp.zeros((128 - d, n), jnp.float32)], axis=0)
    k = _make_seg4(n, e, 64, 2, True)
    outs = []
    for p in range(2):
        sl = xT[p * 64:(p + 1) * 64].reshape(-1)
        outs.append(k(sl, src, dst))
    res = []
    for i in range(4):
        full = jnp.concatenate(
            [outs[0][i].reshape(64, n), outs[1][i].reshape(64, n)], axis=0)
        res.append(full[:d].T)
    return res


def _head_body(x_ref, batch_ref, wl_ref, bl_ref, o_ref):
    x = x_ref[...]
    n = x.shape[0]
    b = batch_ref[...].reshape(n, 1)
    gids = jax.lax.broadcasted_iota(jnp.int32, (n, NGROUP), 1)
    onehot = (b == gids).astype(jnp.float32)
    cnt = jnp.sum(onehot, axis=0)
    pooled = jax.lax.dot_general(onehot, x, (((0,), (0,)), ((), ())))
    pooled = pooled / jnp.clip(cnt, 1.0, None)[:, None]
    out = pooled @ wl_ref[...] + bl_ref[...]
    out = out - jnp.max(out, axis=-1, keepdims=True)
    ex = jnp.exp(out)
    o_ref[...] = ex / jnp.sum(ex, axis=-1, keepdims=True)


def _head(x, batch, Wl, bl):
    return pl.pallas_call(
        _head_body,
        out_shape=jax.ShapeDtypeStruct((NGROUP, Wl.shape[1]), jnp.float32),
    )(x, batch, Wl, bl)


def kernel(x, edge_index, edge_attr, batch, W0, b0, W1, b1, g1, beta1, W2, b2, g2, beta2, W3, b3, g3, beta3, W4, b4, g4, beta4, Wl, bl):
    src, dst = edge_index[0], edge_index[1]
    n, f_in = x.shape
    e = src.shape[0]

    eaT = jnp.concatenate([
        edge_attr.T,
        jnp.ones((1, e), jnp.float32),
        jnp.zeros((NW - 17, e), jnp.float32),
    ], axis=0)
    ea_k = _make_seg4(n, e, NW, 1, False)
    ea_s, ea_q, ea_mn, ea_mx = (o.reshape(NW, n) for o in ea_k(eaT.reshape(-1), src, dst))
    count = ea_s[16]

    deg = jnp.clip(count, 1.0, None)
    inv = 1.0 / deg
    hasm = (count > 0)[:, None]
    dl = jnp.log(deg + 1.0)
    s1 = (dl / AVG)[:, None]
    s2 = (AVG / dl)[:, None]

    ea_mean = ea_s[:16].T * inv[:, None]
    ea_std = jnp.sqrt(jax.nn.relu(ea_q[:16].T * inv[:, None] - ea_mean**2) + 1e-5)
    ea_mn = jnp.where(hasm, ea_mn[:16].T, 0.0)
    ea_mx = jnp.where(hasm, ea_mx[:16].T, 0.0)
    ea_agg = jnp.concatenate([ea_mean, ea_mn, ea_mx, ea_std], axis=-1)

    def layer(xc, xcT, W, b):
        D = xc.shape[1]
        Dt = 2 * D + 16
        s, sq, mn, mx = _x_stats(xcT, src, dst, D)
        mean_src = s * inv[:, None]
        std_src = jnp.sqrt(jax.nn.relu(sq * inv[:, None] - mean_src**2) + 1e-5)
        mn = jnp.where(hasm, mn, 0.0)
        mx = jnp.where(hasm, mx, 0.0)
        xt = jnp.where(hasm, xc, 0.0)
        y = 0.0
        for k in range(3):
            Wk = W[k * 4 * Dt:(k + 1) * 4 * Dt]
            Wd = Wk[0:D] + Wk[Dt:Dt + D] + Wk[2 * Dt:2 * Dt + D]
            yk = xt @ Wd + CSTD * jnp.sum(Wk[3 * Dt:3 * Dt + D], axis=0)
            yk += mean_src @ Wk[D:2 * D] + mn @ Wk[Dt + D:Dt + 2 * D]
            yk += mx @ Wk[2 * Dt + D:2 * Dt + 2 * D] + std_src @ Wk[3 * Dt + D:3 * Dt + 2 * D]
            Wea = jnp.concatenate([Wk[2 * D:2 * D + 16], Wk[Dt + 2 * D:Dt + 2 * D + 16],
                                   Wk[2 * Dt + 2 * D:2 * Dt + 2 * D + 16], Wk[3 * Dt + 2 * D:3 * Dt + 2 * D + 16]], axis=0)
            yk += ea_agg @ Wea
            scale = 1.0 if k == 0 else (s1 if k == 1 else s2)
            y = y + yk * scale
        return y + b

    xc = layer(x, x.T, W0, b0)
    for W, bb, g, be in ((W1, b1, g1, beta1), (W2, b2, g2, beta2), (W3, b3, g3, beta3), (W4, b4, g4, beta4)):
        y = layer(xc, xc.T, W, bb)
        m = jnp.mean(y, axis=0)
        v = jnp.var(y, axis=0)
        h = jax.nn.relu((y - m) / jnp.sqrt(v + 1e-5) * g + be)
        xc = h + xc
    return _head(xc, batch, Wl, bl)


# speedup vs baseline: 3.2343x; 1.1281x over previous
import functools
import jax
import jax.numpy as jnp
import numpy as np
from jax import lax
from jax.experimental import pallas as pl
from jax.experimental.pallas import tpu as pltpu
from jax.experimental.pallas import tpu_sc as plsc

AVG = float(np.log(33.0))
CSTD = float(np.sqrt(1e-5))
NGROUP = 64
NC, NS = 2, 16
NW = NC * NS
BWIN = 2560
UNR = 2


@functools.lru_cache(maxsize=None)
def _make_seg4(n, e, nf, cpw, gather):
    assert nf == NW * cpw and e % BWIN == 0 and n % 16 == 0
    assert gather or cpw == 1
    nwin = e // BWIN
    nvr = BWIN // 16
    mesh = plsc.VectorSubcoreMesh(core_axis_name="c", subcore_axis_name="s")
    out_sds = jax.ShapeDtypeStruct((nf * n,), jnp.float32)

    @functools.partial(
        pl.kernel,
        out_type=(out_sds,) * 4,
        mesh=mesh,
        compiler_params=pltpu.CompilerParams(needs_layout_passes=False),
        scratch_types=(
            [pltpu.VMEM((n,) if gather else (BWIN,), jnp.float32)
             for _ in range(cpw if gather else 1)]
            + [pltpu.VMEM((BWIN,), jnp.int32)] * 2
            + [pltpu.VMEM((n,), jnp.float32) for _ in range(4 * cpw)]
            + [pltpu.VMEM((n,), jnp.int32)] * 2
        ),
    )
    def seg4(vals_hbm, src_hbm, dst_hbm, s_out, q_out, mn_out, mx_out, *refs):
        nv = cpw if gather else 1
        vals_vs = refs[:nv]
        srcv, dstv = refs[nv], refs[nv + 1]
        accs = refs[nv + 2:nv + 2 + 4 * cpw]
        tmps = refs[nv + 2 + 4 * cpw:]
        cid = lax.axis_index("c")
        sid = lax.axis_index("s")
        wid = sid * NC + cid
        lanes = lax.iota(jnp.int32, 16)
        pw2 = jnp.left_shift(jnp.ones((16,), jnp.int32), lanes)
        if gather:
            for r in range(cpw):
                pltpu.sync_copy(
                    vals_hbm.at[pl.ds((wid * cpw + r) * n, n)], vals_vs[r])

        zero = jnp.zeros((16,), jnp.float32)
        pinf = jnp.full((16,), jnp.inf, jnp.float32)
        ninf = jnp.full((16,), -jnp.inf, jnp.float32)
        inits = [zero, zero, pinf, ninf]

        def initb(i, _):
            sl = pl.ds(i * 16, 16)
            for st in range(4):
                for r in range(cpw):
                    accs[st * cpw + r][sl] = inits[st]
            return 0

        lax.fori_loop(0, n // 16, initb, 0)

        def wloop(w, _):
            if gather:
                pltpu.sync_copy(src_hbm.at[pl.ds(w * BWIN, BWIN)], srcv)
            else:
                pltpu.sync_copy(
                    vals_hbm.at[pl.ds(wid * e + w * BWIN, BWIN)], vals_vs[0])
            pltpu.sync_copy(dst_hbm.at[pl.ds(w * BWIN, BWIN)], dstv)

            ids = [lanes + u * 16 for u in range(UNR)]

            def block(jb, tmp):
                base = jb * UNR * 16
                d_idx = [dstv[pl.ds(base + u * 16, 16)] for u in range(UNR)]
                if gather:
                    s_idx = [srcv[pl.ds(base + u * 16, 16)] for u in range(UNR)]
                else:
                    v0 = [vals_vs[0][pl.ds(base + u * 16, 16)] for u in range(UNR)]

                def body(bits):
                    rems = [(jnp.full((16,), bits[u], jnp.int32) & pw2) != 0
                            for u in range(UNR)]
                    for u in range(UNR):
                        plsc.store_scatter(tmp, [d_idx[u]], ids[u], mask=rems[u])
                    wins = []
                    for u in range(UNR):
                        got = plsc.load_gather(tmp, [d_idx[u]])
                        wins.append(rems[u] & (got == ids[u]))
                    for u in range(UNR):
                        for r in range(cpw):
                            if gather:
                                v = plsc.load_gather(vals_vs[r], [s_idx[u]])
                            else:
                                v = v0[u]
                            a = d_idx[u]
                            win = wins[u]
                            plsc.addupdate_scatter(accs[r], [a], v, mask=win)
                            plsc.addupdate_scatter(accs[cpw + r], [a], v * v, mask=win)
                            for st in (2, 3):
                                acc = accs[st * cpw + r]
                                c0 = plsc.load_gather(acc, [a])
                                nv = jnp.minimum(c0, v) if st == 2 else jnp.maximum(c0, v)
                                plsc.store_scatter(acc, [a], nv, mask=win)
                    return tuple(
                        bits[u] & jnp.bitwise_not(jnp.sum(jnp.where(wins[u], pw2, 0)))
                        for u in range(UNR))

                bits1 = body((jnp.int32(0xFFFF),) * UNR)
                anyb = bits1[0]
                for u in range(1, UNR):
                    anyb = anyb | bits1[u]

                @pl.when(anyb != 0)
                def _():
                    def retry(i, b):
                        run = b[0]
                        for u in range(1, UNR):
                            run = run | b[u]
                        return lax.cond(run != 0, body, lambda bb: bb, b)

                    lax.fori_loop(0, 15, retry, bits1)

            def vrb(jp, _):
                block(jp * 2, tmps[0])
                block(jp * 2 + 1, tmps[1])
                return 0

            lax.fori_loop(0, nvr // UNR // 2, vrb, 0)
            return 0

        lax.fori_loop(0, nwin, wloop, 0)

        for st, out in enumerate((s_out, q_out, mn_out, mx_out)):
            for r in range(cpw):
                pltpu.sync_copy(
                    accs[st * cpw + r],
                    out.at[pl.ds((wid * cpw + r) * n, n)])

    return seg4


def _x_stats(xT, src, dst, d):
    n = xT.shape[1]
    e = src.shape[0]
    outs = []
    widths = []
    p = 0
    while p < d:
        w = 64 if d - p > 32 else 32
        cpw = w // NW
        kfn = _make_seg4(n, e, w, cpw, True)
        blk = xT[p:p + w]
        if blk.shape[0] < w:
            blk = jnp.concatenate(
                [blk, jnp.zeros((w - blk.shape[0], n), jnp.float32)], axis=0)
        outs.append((kfn(blk.reshape(-1), src, dst), w))
        p += w
    res = []
    for i in range(4):
        full = jnp.concatenate(
            [o[i].reshape(w, n) for o, w in outs], axis=0)
        res.append(full[:d].T)
    return res


def _head_body(x_ref, batch_ref, wl_ref, bl_ref, o_ref):
    x = x_ref[...]
    n = x.shape[0]
    b = batch_ref[...].reshape(n, 1)
    gids = jax.lax.broadcasted_iota(jnp.int32, (n, NGROUP), 1)
    onehot = (b == gids).astype(jnp.float32)
    cnt = jnp.sum(onehot, axis=0)
    pooled = jax.lax.dot_general(onehot, x, (((0,), (0,)), ((), ())))
    pooled = pooled / jnp.clip(cnt, 1.0, None)[:, None]
    out = pooled @ wl_ref[...] + bl_ref[...]
    out = out - jnp.max(out, axis=-1, keepdims=True)
    ex = jnp.exp(out)
    o_ref[...] = ex / jnp.sum(ex, axis=-1, keepdims=True)


def _head(x, batch, Wl, bl):
    return pl.pallas_call(
        _head_body,
        out_shape=jax.ShapeDtypeStruct((NGROUP, Wl.shape[1]), jnp.float32),
    )(x, batch, Wl, bl)


def kernel(x, edge_index, edge_attr, batch, W0, b0, W1, b1, g1, beta1, W2, b2, g2, beta2, W3, b3, g3, beta3, W4, b4, g4, beta4, Wl, bl):
    src, dst = edge_index[0], edge_index[1]
    n, f_in = x.shape
    e = src.shape[0]

    eaT = jnp.concatenate([
        edge_attr.T,
        jnp.ones((1, e), jnp.float32),
        jnp.zeros((NW - 17, e), jnp.float32),
    ], axis=0)
    ea_k = _make_seg4(n, e, NW, 1, False)
    ea_s, ea_q, ea_mn, ea_mx = (o.reshape(NW, n) for o in ea_k(eaT.reshape(-1), src, dst))
    count = ea_s[16]

    deg = jnp.clip(count, 1.0, None)
    inv = 1.0 / deg
    hasm = (count > 0)[:, None]
    dl = jnp.log(deg + 1.0)
    s1 = (dl / AVG)[:, None]
    s2 = (AVG / dl)[:, None]

    ea_mean = ea_s[:16].T * inv[:, None]
    ea_std = jnp.sqrt(jax.nn.relu(ea_q[:16].T * inv[:, None] - ea_mean**2) + 1e-5)
    ea_mn = jnp.where(hasm, ea_mn[:16].T, 0.0)
    ea_mx = jnp.where(hasm, ea_mx[:16].T, 0.0)
    ea_agg = jnp.concatenate([ea_mean, ea_mn, ea_mx, ea_std], axis=-1)

    def layer(xc, xcT, W, b):
        D = xc.shape[1]
        Dt = 2 * D + 16
        s, sq, mn, mx = _x_stats(xcT, src, dst, D)
        mean_src = s * inv[:, None]
        std_src = jnp.sqrt(jax.nn.relu(sq * inv[:, None] - mean_src**2) + 1e-5)
        mn = jnp.where(hasm, mn, 0.0)
        mx = jnp.where(hasm, mx, 0.0)
        xt = jnp.where(hasm, xc, 0.0)
        y = 0.0
        for k in range(3):
            Wk = W[k * 4 * Dt:(k + 1) * 4 * Dt]
            Wd = Wk[0:D] + Wk[Dt:Dt + D] + Wk[2 * Dt:2 * Dt + D]
            yk = xt @ Wd + CSTD * jnp.sum(Wk[3 * Dt:3 * Dt + D], axis=0)
            yk += mean_src @ Wk[D:2 * D] + mn @ Wk[Dt + D:Dt + 2 * D]
            yk += mx @ Wk[2 * Dt + D:2 * Dt + 2 * D] + std_src @ Wk[3 * Dt + D:3 * Dt + 2 * D]
            Wea = jnp.concatenate([Wk[2 * D:2 * D + 16], Wk[Dt + 2 * D:Dt + 2 * D + 16],
                                   Wk[2 * Dt + 2 * D:2 * Dt + 2 * D + 16], Wk[3 * Dt + 2 * D:3 * Dt + 2 * D + 16]], axis=0)
            yk += ea_agg @ Wea
            scale = 1.0 if k == 0 else (s1 if k == 1 else s2)
            y = y + yk * scale
        return y + b

    xc = layer(x, x.T, W0, b0)
    for W, bb, g, be in ((W1, b1, g1, beta1), (W2, b2, g2, beta2), (W3, b3, g3, beta3), (W4, b4, g4, beta4)):
        y = layer(xc, xc.T, W, bb)
        m = jnp.mean(y, axis=0)
        v = jnp.var(y, axis=0)
        h = jax.nn.relu((y - m) / jnp.sqrt(v + 1e-5) * g + be)
        xc = h + xc
    return _head(xc, batch, Wl, bl)
